# baseline (speedup 1.0000x reference)
import jax
import jax.numpy as jnp
from jax import lax
from jax.experimental import pallas as pl
from jax.experimental.pallas import tpu as pltpu
from jax.experimental.pallas import tpu_sc as plsc

_VOCAB = 100000
_BUCKETS = 100000
_EMB = 64
_BATCH = 4096
_HIST = 20

_NC = 2
_NS = 16
_NW = _NC * _NS
_DPW = _EMB // _NW

_HB = _BATCH // 2
_STRIPE = 512
_NSTRIPE = _HB // _STRIPE
_BLK = _STRIPE // 16


def _pool_body(target_hbm, ngrams_t_hbm, emb_t_hbm, ngram_emb_t_hbm, vt_hbm,
               tidx, nidx, plane, acc, sem):
    wid = lax.axis_index("s") * _NC + lax.axis_index("c")
    pltpu.sync_copy(target_hbm, tidx)
    inv = jnp.float32(1.0 / _HIST)

    for d in range(_DPW):
        e = wid * _DPW + d

        pltpu.sync_copy(ngram_emb_t_hbm.at[e], plane)

        def stripe_body(st, c):
            base = st * _STRIPE
            pltpu.sync_copy(ngrams_t_hbm.at[:, pl.ds(base, _STRIPE)], nidx)

            def bblock(bi, c2):
                s = pl.ds(bi * 16, 16)
                part = [plsc.load_gather(plane, [nidx[j, s]]) for j in range(4)]
                for j in range(4, _HIST):
                    part[j % 4] = part[j % 4] + plsc.load_gather(plane, [nidx[j, s]])
                acc[pl.ds(base + bi * 16, 16)] = (part[0] + part[1]) + (part[2] + part[3])
                return c2
            lax.fori_loop(0, _BLK, bblock, 0)
            return c
        lax.fori_loop(0, _NSTRIPE, stripe_body, 0)

        pltpu.sync_copy(emb_t_hbm.at[e], plane)

        def fblock(bi, c):
            s = pl.ds(bi * 16, 16)
            acc[s] = acc[s] * inv + plsc.load_gather(plane, [tidx[s]])
            return c
        lax.fori_loop(0, _HB // 16, fblock, 0)

        pltpu.sync_copy(acc, vt_hbm.at[e])


_pool = pl.kernel(
    _pool_body,
    out_type=jax.ShapeDtypeStruct((_EMB, _HB), jnp.float32),
    mesh=plsc.VectorSubcoreMesh(core_axis_name="c", subcore_axis_name="s",
                                num_cores=_NC, num_subcores=_NS),
    scratch_types=[
        pltpu.VMEM((_HB,), jnp.int32),
        pltpu.VMEM((_HIST, _STRIPE), jnp.int32),
        pltpu.VMEM((_BUCKETS,), jnp.float32),
        pltpu.VMEM((_HB,), jnp.float32),
        pltpu.SemaphoreType.DMA,
    ],
    compiler_params=pltpu.CompilerParams(needs_layout_passes=False),
)


_BN = 1024


def _mm_body(wt_ref, vt_ref, b_ref, o_ref):
    o_ref[...] = lax.dot_general(
        wt_ref[...], vt_ref[...], (((0,), (0,)), ((), ())),
        preferred_element_type=jnp.float32) + b_ref[...]


def _mm_body_alias(wt_ref, vt_ref, b_ref, _prev_ref, o_ref):
    _mm_body(wt_ref, vt_ref, b_ref, o_ref)


def _project_half(Wt, vt, bc, half, prev=None):
    nv = pl.cdiv(_VOCAB, _BN)
    in_specs = [
        pl.BlockSpec((_EMB, _BN), lambda i: (0, i)),
        pl.BlockSpec((_EMB, _HB), lambda i: (0, 0)),
        pl.BlockSpec((_BN, 1), lambda i: (i, 0)),
    ]
    args = [Wt, vt, bc]
    kwargs = {}
    body = _mm_body
    if prev is not None:
        in_specs.append(pl.BlockSpec(memory_space=pl.ANY))
        args.append(prev)
        kwargs["input_output_aliases"] = {3: 0}
        body = _mm_body_alias
    return pl.pallas_call(
        body,
        grid=(nv,),
        in_specs=in_specs,
        out_specs=pl.BlockSpec((_BN, _HB), lambda i, h=half: (i, h)),
        out_shape=jax.ShapeDtypeStruct((_VOCAB, _BATCH), jnp.float32),
        compiler_params=pltpu.CompilerParams(
            dimension_semantics=("arbitrary",)),
        **kwargs,
    )(*args)


def kernel(target, ngrams, emb, ngram_emb, W, b):
    target = target.astype(jnp.int32)
    ngrams_t = ngrams.astype(jnp.int32).T
    emb_t, ngram_emb_t = emb.T, ngram_emb.T
    Wt, bc = W.T, b.reshape(_VOCAB, 1)

    vt0 = _pool(target[:_HB], ngrams_t[:, :_HB], emb_t, ngram_emb_t)
    vt1 = _pool(target[_HB:], ngrams_t[:, _HB:], emb_t, ngram_emb_t)
    ot = _project_half(Wt, vt0, bc, 0)
    ot = _project_half(Wt, vt1, bc, 1, prev=ot)
    return ot.T


# speedup vs baseline: 1.0978x; 1.0978x over previous
import functools

import jax
import jax.numpy as jnp
from jax import lax
from jax.experimental import pallas as pl
from jax.experimental.pallas import tpu as pltpu
from jax.experimental.pallas import tpu_sc as plsc

_VOCAB = 100000
_BUCKETS = 100000
_EMB = 64
_BATCH = 4096
_HIST = 20

_NC = 2
_NS = 16
_NW = _NC * _NS
_BPW = _BATCH // _NW
_LANE_CHUNKS = _EMB // 16


_DPW = _EMB // _NW
_STRIPE = 512
_NSTRIPE = _BATCH // _STRIPE
_BLK = _STRIPE // 16


def _pool_body(target_hbm, ngrams_t_hbm, emb_t_hbm, ngram_emb_t_hbm, vt_hbm,
               tidx, nidx, plane, acc, sem):
    wid = lax.axis_index("s") * _NC + lax.axis_index("c")
    pltpu.sync_copy(target_hbm, tidx)
    inv = jnp.float32(1.0 / _HIST)

    for d in range(_DPW):
        e = wid * _DPW + d

        pltpu.sync_copy(ngram_emb_t_hbm.at[e], plane)

        def stripe_body(st, c):
            base = st * _STRIPE
            pltpu.sync_copy(ngrams_t_hbm.at[:, pl.ds(base, _STRIPE)], nidx)

            def bblock(bi, c2):
                s = pl.ds(bi * 16, 16)
                part = [plsc.load_gather(plane, [nidx[j, s]]) for j in range(4)]
                for j in range(4, _HIST):
                    part[j % 4] = part[j % 4] + plsc.load_gather(plane, [nidx[j, s]])
                acc[pl.ds(base + bi * 16, 16)] = (part[0] + part[1]) + (part[2] + part[3])
                return c2
            lax.fori_loop(0, _BLK, bblock, 0)
            return c
        lax.fori_loop(0, _NSTRIPE, stripe_body, 0)

        pltpu.sync_copy(emb_t_hbm.at[e], plane)

        def fblock(bi, c):
            s = pl.ds(bi * 16, 16)
            acc[s] = acc[s] * inv + plsc.load_gather(plane, [tidx[s]])
            return c
        lax.fori_loop(0, _BATCH // 16, fblock, 0)

        pltpu.sync_copy(acc, vt_hbm.at[e])


_pool = pl.kernel(
    _pool_body,
    out_type=jax.ShapeDtypeStruct((_EMB, _BATCH), jnp.float32),
    mesh=plsc.VectorSubcoreMesh(core_axis_name="c", subcore_axis_name="s",
                                num_cores=_NC, num_subcores=_NS),
    scratch_types=[
        pltpu.VMEM((_BATCH,), jnp.int32),
        pltpu.VMEM((_HIST, _STRIPE), jnp.int32),
        pltpu.VMEM((_BUCKETS,), jnp.float32),
        pltpu.VMEM((_BATCH,), jnp.float32),
        pltpu.SemaphoreType.DMA,
    ],
    compiler_params=pltpu.CompilerParams(needs_layout_passes=False),
)


_BN = 1792


def _mm_body(wt_ref, vt_ref, b_ref, o_ref):
    o_ref[...] = lax.dot_general(
        wt_ref[...], vt_ref[...], (((0,), (0,)), ((), ())),
        preferred_element_type=jnp.float32) + b_ref[...]


def _project_t(Wt, vt, bc):
    nv = pl.cdiv(_VOCAB, _BN)
    return pl.pallas_call(
        _mm_body,
        grid=(nv,),
        in_specs=[
            pl.BlockSpec((_EMB, _BN), lambda i: (0, i)),
            pl.BlockSpec((_EMB, _BATCH), lambda i: (0, 0)),
            pl.BlockSpec((_BN, 1), lambda i: (i, 0)),
        ],
        out_specs=pl.BlockSpec((_BN, _BATCH), lambda i: (i, 0)),
        out_shape=jax.ShapeDtypeStruct((_VOCAB, _BATCH), jnp.float32),
        compiler_params=pltpu.CompilerParams(
            dimension_semantics=("arbitrary",),
            vmem_limit_bytes=112 * 1024 * 1024),
    )(Wt, vt, bc)


def kernel(target, ngrams, emb, ngram_emb, W, b):
    target = target.astype(jnp.int32)
    ngrams_t = ngrams.astype(jnp.int32).T
    vt = _pool(target, ngrams_t, emb.T, ngram_emb.T)
    ot = _project_t(W.T, vt, b.reshape(_VOCAB, 1))
    return ot.T


# speedup vs baseline: 1.0996x; 1.0016x over previous
import functools

import jax
import jax.numpy as jnp
from jax import lax
from jax.experimental import pallas as pl
from jax.experimental.pallas import tpu as pltpu
from jax.experimental.pallas import tpu_sc as plsc

_VOCAB = 100000
_BUCKETS = 100000
_EMB = 64
_BATCH = 4096
_HIST = 20

_NC = 2
_NS = 16
_NW = _NC * _NS
_BPW = _BATCH // _NW
_LANE_CHUNKS = _EMB // 16


_DPW = _EMB // _NW
_STRIPE = 256
_NSTRIPE = _BATCH // _STRIPE
_BLK = _STRIPE // 16


def _pool_body(target_hbm, ngrams_t_hbm, emb_t_hbm, ngram_emb_t_hbm, vt_hbm,
               tidx, nidx0, nidx1, plane, acc, sem, semi0, semi1):
    wid = lax.axis_index("s") * _NC + lax.axis_index("c")
    pltpu.sync_copy(target_hbm, tidx)
    inv = jnp.float32(1.0 / _HIST)
    nbufs = (nidx0, nidx1)
    sems = (semi0, semi1)

    for d in range(_DPW):
        e = wid * _DPW + d

        pcopy = pltpu.async_copy(ngram_emb_t_hbm.at[e], plane, sem)
        cps = [pltpu.async_copy(
            ngrams_t_hbm.at[:, pl.ds(st * _STRIPE, _STRIPE)], nbufs[st % 2],
            sems[st % 2]) for st in range(2)]
        pcopy.wait()
        for st in range(_NSTRIPE):
            nidx = nbufs[st % 2]
            base = st * _STRIPE
            cps[st % 2].wait()

            def bblock(bi, c2, nidx=nidx, base=base):
                s = pl.ds(bi * 16, 16)
                part = [plsc.load_gather(plane, [nidx[j, s]]) for j in range(4)]
                for j in range(4, _HIST):
                    part[j % 4] = part[j % 4] + plsc.load_gather(plane, [nidx[j, s]])
                acc[pl.ds(base + bi * 16, 16)] = (part[0] + part[1]) + (part[2] + part[3])
                return c2
            lax.fori_loop(0, _BLK, bblock, 0)
            if st + 2 < _NSTRIPE:
                cps[st % 2] = pltpu.async_copy(
                    ngrams_t_hbm.at[:, pl.ds((st + 2) * _STRIPE, _STRIPE)],
                    nbufs[st % 2], sems[st % 2])

        pltpu.sync_copy(emb_t_hbm.at[e], plane)

        def fblock(bi, c):
            s = pl.ds(bi * 16, 16)
            acc[s] = acc[s] * inv + plsc.load_gather(plane, [tidx[s]])
            return c
        lax.fori_loop(0, _BATCH // 16, fblock, 0)

        pltpu.sync_copy(acc, vt_hbm.at[e])


_pool = pl.kernel(
    _pool_body,
    out_type=jax.ShapeDtypeStruct((_EMB, _BATCH), jnp.float32),
    mesh=plsc.VectorSubcoreMesh(core_axis_name="c", subcore_axis_name="s",
                                num_cores=_NC, num_subcores=_NS),
    scratch_types=[
        pltpu.VMEM((_BATCH,), jnp.int32),
        pltpu.VMEM((_HIST, _STRIPE), jnp.int32),
        pltpu.VMEM((_HIST, _STRIPE), jnp.int32),
        pltpu.VMEM((_BUCKETS,), jnp.float32),
        pltpu.VMEM((_BATCH,), jnp.float32),
        pltpu.SemaphoreType.DMA,
        pltpu.SemaphoreType.DMA,
        pltpu.SemaphoreType.DMA,
    ],
    compiler_params=pltpu.CompilerParams(needs_layout_passes=False),
)


_BN = 1792


def _mm_body(wt_ref, vt_ref, b_ref, o_ref):
    o_ref[...] = lax.dot_general(
        wt_ref[...], vt_ref[...], (((0,), (0,)), ((), ())),
        preferred_element_type=jnp.float32) + b_ref[...]


def _project_t(Wt, vt, bc):
    nv = pl.cdiv(_VOCAB, _BN)
    return pl.pallas_call(
        _mm_body,
        grid=(nv,),
        in_specs=[
            pl.BlockSpec((_EMB, _BN), lambda i: (0, i)),
            pl.BlockSpec((_EMB, _BATCH), lambda i: (0, 0)),
            pl.BlockSpec((_BN, 1), lambda i: (i, 0)),
        ],
        out_specs=pl.BlockSpec((_BN, _BATCH), lambda i: (i, 0)),
        out_shape=jax.ShapeDtypeStruct((_VOCAB, _BATCH), jnp.float32),
        compiler_params=pltpu.CompilerParams(
            dimension_semantics=("arbitrary",),
            vmem_limit_bytes=112 * 1024 * 1024),
    )(Wt, vt, bc)


def kernel(target, ngrams, emb, ngram_emb, W, b):
    target = target.astype(jnp.int32)
    ngrams_t = ngrams.astype(jnp.int32).T
    vt = _pool(target, ngrams_t, emb.T, ngram_emb.T)
    ot = _project_t(W.T, vt, b.reshape(_VOCAB, 1))
    return ot.T


# speedup vs baseline: 1.1001x; 1.0005x over previous
import jax
import jax.numpy as jnp
from jax import lax
from jax.experimental import pallas as pl
from jax.experimental.pallas import tpu as pltpu
from jax.experimental.pallas import tpu_sc as plsc

_VOCAB = 100000
_BUCKETS = 100000
_EMB = 64
_BATCH = 4096
_HIST = 20

_NC = 2
_NS = 16
_NW = _NC * _NS


_DPW = _EMB // _NW
_STRIPE = 256
_NSTRIPE = _BATCH // _STRIPE
_BLK = _STRIPE // 16


def _pool_body(target_hbm, ngrams_t_hbm, emb_t_hbm, ngram_emb_t_hbm, vt_hbm,
               tidx, nidx0, nidx1, plane, acc, sem, semi0, semi1):
    wid = lax.axis_index("s") * _NC + lax.axis_index("c")
    pltpu.sync_copy(target_hbm, tidx)
    inv = jnp.float32(1.0 / _HIST)
    nbufs = (nidx0, nidx1)
    sems = (semi0, semi1)

    for d in range(_DPW):
        e = wid * _DPW + d

        pcopy = pltpu.async_copy(ngram_emb_t_hbm.at[e], plane, sem)
        cps = [pltpu.async_copy(
            ngrams_t_hbm.at[:, pl.ds(st * _STRIPE, _STRIPE)], nbufs[st % 2],
            sems[st % 2]) for st in range(2)]
        pcopy.wait()
        for st in range(_NSTRIPE):
            nidx = nbufs[st % 2]
            base = st * _STRIPE
            cps[st % 2].wait()

            def bblock(bi, c2, nidx=nidx, base=base):
                s = pl.ds(bi * 16, 16)
                part = [plsc.load_gather(plane, [nidx[j, s]]) for j in range(4)]
                for j in range(4, _HIST):
                    part[j % 4] = part[j % 4] + plsc.load_gather(plane, [nidx[j, s]])
                acc[pl.ds(base + bi * 16, 16)] = (part[0] + part[1]) + (part[2] + part[3])
                return c2
            lax.fori_loop(0, _BLK, bblock, 0)
            if st + 2 < _NSTRIPE:
                cps[st % 2] = pltpu.async_copy(
                    ngrams_t_hbm.at[:, pl.ds((st + 2) * _STRIPE, _STRIPE)],
                    nbufs[st % 2], sems[st % 2])

        pltpu.sync_copy(emb_t_hbm.at[e], plane)

        def fblock(bi, c):
            s = pl.ds(bi * 16, 16)
            acc[s] = acc[s] * inv + plsc.load_gather(plane, [tidx[s]])
            return c
        lax.fori_loop(0, _BATCH // 16, fblock, 0)

        pltpu.sync_copy(acc, vt_hbm.at[e])


_pool = pl.kernel(
    _pool_body,
    out_type=jax.ShapeDtypeStruct((_EMB, _BATCH), jnp.float32),
    mesh=plsc.VectorSubcoreMesh(core_axis_name="c", subcore_axis_name="s",
                                num_cores=_NC, num_subcores=_NS),
    scratch_types=[
        pltpu.VMEM((_BATCH,), jnp.int32),
        pltpu.VMEM((_HIST, _STRIPE), jnp.int32),
        pltpu.VMEM((_HIST, _STRIPE), jnp.int32),
        pltpu.VMEM((_BUCKETS,), jnp.float32),
        pltpu.VMEM((_BATCH,), jnp.float32),
        pltpu.SemaphoreType.DMA,
        pltpu.SemaphoreType.DMA,
        pltpu.SemaphoreType.DMA,
    ],
    compiler_params=pltpu.CompilerParams(needs_layout_passes=False),
)


_BN = 1792


def _mm_body(wt_ref, vt_ref, b_ref, o_ref):
    o_ref[...] = lax.dot_general(
        wt_ref[...], vt_ref[...], (((0,), (0,)), ((), ())),
        preferred_element_type=jnp.float32) + b_ref[...]


def _project_t(Wt, vt, bc):
    nv = pl.cdiv(_VOCAB, _BN)
    return pl.pallas_call(
        _mm_body,
        grid=(nv,),
        in_specs=[
            pl.BlockSpec((_EMB, _BN), lambda i: (0, i)),
            pl.BlockSpec((_EMB, _BATCH), lambda i: (0, 0)),
            pl.BlockSpec((_BN, 1), lambda i: (i, 0)),
        ],
        out_specs=pl.BlockSpec((_BN, _BATCH), lambda i: (i, 0)),
        out_shape=jax.ShapeDtypeStruct((_VOCAB, _BATCH), jnp.float32),
        compiler_params=pltpu.CompilerParams(
            dimension_semantics=("arbitrary",),
            vmem_limit_bytes=112 * 1024 * 1024),
    )(Wt, vt, bc)


def kernel(target, ngrams, emb, ngram_emb, W, b):
    target = target.astype(jnp.int32)
    ngrams_t = ngrams.astype(jnp.int32).T
    vt = _pool(target, ngrams_t, emb.T, ngram_emb.T)
    ot = _project_t(W.T, vt, b.reshape(_VOCAB, 1))
    return ot.T
